# baseline (speedup 1.0000x reference)
import functools

import jax
import jax.numpy as jnp
from jax import lax
from jax.experimental import pallas as pl
from jax.experimental.pallas import tpu as pltpu
from jax.experimental.pallas import tpu_sc as plsc

BATCH = 16384
D = 32
NROWS = 1000000
PACK = 4
BN_EPS = 1e-5

NC = 2
NS = 16
NW = NC * NS
BPW = BATCH // NW
CH = 128
NCH = BPW // CH

RB = 512
UB = RB * PACK
RGRID = -(-NROWS // UB)
PROWS = RGRID * RB


def _repack_body(*refs):
    ins, outs = refs[:4], refs[4:]
    eye = jnp.eye(D, dtype=jnp.float32)
    hi = jax.lax.Precision.HIGHEST
    for t in range(4):
        x = ins[t][...]
        xt = jax.lax.dot_general(x, eye, (((0,), (0,)), ((), ())),
                                 preferred_element_type=jnp.float32,
                                 precision=hi)
        cols = []
        for a in range(PACK):
            rows = [xt[512 * gl + 128 * a:512 * gl + 128 * a + 128]
                    for gl in range(RB // 128)]
            cols.append(jnp.concatenate(rows, axis=0))
        outs[t][...] = jnp.concatenate(cols, axis=1)


def _tc_repack(tablesT):
    in_specs = [pl.BlockSpec((D, UB), lambda i: (0, i)) for _ in range(4)]
    out_specs = [pl.BlockSpec((RB, PACK * D), lambda i: (i, 0))] * 4
    out_shape = [jax.ShapeDtypeStruct((PROWS, PACK * D), jnp.float32)] * 4
    return pl.pallas_call(
        _repack_body,
        grid=(RGRID,),
        in_specs=in_specs,
        out_specs=out_specs,
        out_shape=out_shape,
    )(*tablesT)


def _sc_gather(ju2, ji2, pug, pig, pum, pim):
    mesh = plsc.VectorSubcoreMesh(core_axis_name="c", subcore_axis_name="s")

    @functools.partial(
        pl.kernel,
        mesh=mesh,
        out_type=[jax.ShapeDtypeStruct((BATCH, PACK * D), jnp.float32)] * 4,
        scratch_types=[
            pltpu.VMEM((NCH, CH), jnp.int32),
            pltpu.VMEM((NCH, CH), jnp.int32),
            pltpu.VMEM((BPW, PACK * D), jnp.float32),
            pltpu.SemaphoreType.DMA,
        ],
    )
    def k(ju_hbm, ji_hbm, t0, t1, t2, t3, o0, o1, o2, o3, uix, iix, buf, sem):
        wid = lax.axis_index("s") * NC + lax.axis_index("c")
        base = wid * BPW
        row0 = wid * NCH
        pltpu.sync_copy(ju_hbm.at[pl.ds(row0, NCH)], uix)
        pltpu.sync_copy(ji_hbm.at[pl.ds(row0, NCH)], iix)
        for tbl, out, idx in ((t0, o0, uix), (t1, o1, iix),
                              (t2, o2, uix), (t3, o3, iix)):
            copies = []
            for j in range(NCH):
                copies.append(pltpu.async_copy(
                    tbl.at[idx.at[j]], buf.at[pl.ds(j * CH, CH)], sem))
            for c in copies:
                c.wait()
            pltpu.sync_copy(buf, out.at[pl.ds(base, BPW)])

    return k(ju2, ji2, pug, pig, pum, pim)


def _select(block, res, k0):
    acc = jnp.zeros((block.shape[0], D), jnp.float32)
    for a in range(PACK):
        acc = acc + jnp.where(res == a, block[:, a * D:(a + 1) * D], 0.0)
    del k0
    return acc


def _dense_body(gug_ref, gig_ref, gum_ref, gim_ref, ru_ref, ri_ref,
                w0u, w0i, b0, w1, b1, w2, b2, w3, b3, wpg, wph, bp,
                out_ref):
    f32 = jnp.float32
    hi = jax.lax.Precision.HIGHEST
    ru = ru_ref[...]
    ri = ri_ref[...]
    ug = _select(gug_ref[...], ru, 0)
    ig = _select(gig_ref[...], ri, 0)
    um = _select(gum_ref[...], ru, 0)
    im = _select(gim_ref[...], ri, 0)
    nu = jnp.sqrt(jnp.sum(ug * ug, axis=1, keepdims=True))
    ni = jnp.sqrt(jnp.sum(ig * ig, axis=1, keepdims=True))
    gmf = (ug / jnp.maximum(nu, 1e-12)) * (ig / jnp.maximum(ni, 1e-12))
    h = (jnp.dot(um, w0u[...], preferred_element_type=f32, precision=hi)
         + jnp.dot(im, w0i[...], preferred_element_type=f32, precision=hi)
         + b0[...])
    h = jnp.maximum(h, 0.0)
    for w, b in ((w1, b1), (w2, b2), (w3, b3)):
        h = jnp.dot(h, w[...], preferred_element_type=f32, precision=hi) + b[...]
        h = jnp.maximum(h, 0.0)
    pred = (jnp.dot(gmf, wpg[...], preferred_element_type=f32, precision=hi)
            + jnp.dot(h, wph[...], preferred_element_type=f32, precision=hi)
            + bp[...])
    out_ref[...] = jax.nn.sigmoid(pred)


def kernel(user_indices, item_indices, user_emb_gmf, item_emb_gmf,
           user_emb_mlp, item_emb_mlp,
           W0, b0, gamma0, beta0, W1, b1, gamma1, beta1,
           W2, b2, gamma2, beta2, W3, b3, gamma3, beta3,
           Wp, bp):
    uidx = user_indices.astype(jnp.int32)
    iidx = item_indices.astype(jnp.int32)
    ju = 128 * (uidx // 512) + uidx % 128
    ji = 128 * (iidx // 512) + iidx % 128
    ju2 = ju.reshape(BATCH // CH, CH)
    ji2 = ji.reshape(BATCH // CH, CH)
    ru = ((uidx // 128) % PACK).astype(jnp.int32).reshape(BATCH, 1)
    ri = ((iidx // 128) % PACK).astype(jnp.int32).reshape(BATCH, 1)

    packed = _tc_repack((user_emb_gmf.T, item_emb_gmf.T,
                         user_emb_mlp.T, item_emb_mlp.T))
    gug, gig, gum, gim = _sc_gather(ju2, ji2, *packed)

    bn = 1.0 / jnp.sqrt(1.0 + BN_EPS)
    def fold(W, b, g, be):
        s = g * bn
        return W * s[None, :], (b * s + be)[None, :]
    W0f, b0f = fold(W0, b0, gamma0, beta0)
    W1f, b1f = fold(W1, b1, gamma1, beta1)
    W2f, b2f = fold(W2, b2, gamma2, beta2)
    W3f, b3f = fold(W3, b3, gamma3, beta3)
    w0u, w0i = W0f[:D], W0f[D:]
    wpg, wph = Wp[:D], Wp[D:]
    bp2 = bp[None, :]

    BB = 2048
    grid = (BATCH // BB,)
    g_spec = pl.BlockSpec((BB, PACK * D), lambda i: (i, 0))
    r_spec = pl.BlockSpec((BB, 1), lambda i: (i, 0))
    def w_spec(a):
        return pl.BlockSpec(a.shape, lambda i: (0,) * a.ndim)
    weights = (w0u, w0i, b0f, W1f, b1f, W2f, b2f, W3f, b3f, wpg, wph, bp2)

    out = pl.pallas_call(
        _dense_body,
        grid=grid,
        in_specs=[g_spec] * 4 + [r_spec] * 2 + [w_spec(a) for a in weights],
        out_specs=pl.BlockSpec((BB, 1), lambda i: (i, 0)),
        out_shape=jax.ShapeDtypeStruct((BATCH, 1), jnp.float32),
    )(gug, gig, gum, gim, ru, ri, *weights)
    return out


# speedup vs baseline: 2.7826x; 2.7826x over previous
import functools

import jax
import jax.numpy as jnp
from jax import lax
from jax.experimental import pallas as pl
from jax.experimental.pallas import tpu as pltpu
from jax.experimental.pallas import tpu_sc as plsc

BATCH = 16384
D = 32
NROWS = 1000000
PACK = 4
BN_EPS = 1e-5

NC = 2
NS = 16
NW = NC * NS
BPW = BATCH // NW
CH = 128
NCH = BPW // CH

RB = 512
UB = RB * PACK
RGRID = -(-NROWS // UB)
PROWS = RGRID * RB


def _repack_body(*refs):
    ins, outs = refs[:4], refs[4:]
    base = jnp.eye(D, dtype=jnp.bfloat16)
    eyes = [jnp.pad(base, ((0, 0), (D * a, D * (PACK - 1 - a))))
            for a in range(PACK)]
    limit = NROWS - pl.program_id(0) * UB
    ok = jax.lax.broadcasted_iota(jnp.int32, (D, UB), 1) < limit
    for t in range(4):
        x = ins[t][...]
        xb = jnp.where(ok, x, 0.0).astype(jnp.bfloat16)
        acc = None
        for a in range(PACK):
            xa = xb[:, 512 * a:512 * (a + 1)]
            p = jax.lax.dot_general(xa, eyes[a], (((0,), (0,)), ((), ())),
                                    preferred_element_type=jnp.float32)
            acc = p if acc is None else acc + p
        outs[t][...] = acc


def _tc_repack(tablesT):
    in_specs = [pl.BlockSpec((D, UB), lambda i: (0, i)) for _ in range(4)]
    out_specs = [pl.BlockSpec((RB, PACK * D), lambda i: (i, 0))] * 4
    out_shape = [jax.ShapeDtypeStruct((PROWS, PACK * D), jnp.float32)] * 4
    return pl.pallas_call(
        _repack_body,
        grid=(RGRID,),
        in_specs=in_specs,
        out_specs=out_specs,
        out_shape=out_shape,
    )(*tablesT)


def _sc_gather(ju2, ji2, pug, pig, pum, pim):
    mesh = plsc.VectorSubcoreMesh(core_axis_name="c", subcore_axis_name="s")

    @functools.partial(
        pl.kernel,
        mesh=mesh,
        out_type=[jax.ShapeDtypeStruct((BATCH, PACK * D), jnp.float32)] * 4,
        scratch_types=[
            pltpu.VMEM((NCH, CH), jnp.int32),
            pltpu.VMEM((NCH, CH), jnp.int32),
            pltpu.VMEM((BPW, PACK * D), jnp.float32),
            pltpu.SemaphoreType.DMA,
        ],
    )
    def k(ju_hbm, ji_hbm, t0, t1, t2, t3, o0, o1, o2, o3, uix, iix, buf, sem):
        wid = lax.axis_index("s") * NC + lax.axis_index("c")
        base = wid * BPW
        row0 = wid * NCH
        pltpu.sync_copy(ju_hbm.at[pl.ds(row0, NCH)], uix)
        pltpu.sync_copy(ji_hbm.at[pl.ds(row0, NCH)], iix)
        for tbl, out, idx in ((t0, o0, uix), (t1, o1, iix),
                              (t2, o2, uix), (t3, o3, iix)):
            copies = []
            for j in range(NCH):
                copies.append(pltpu.async_copy(
                    tbl.at[idx.at[j]], buf.at[pl.ds(j * CH, CH)], sem))
            for c in copies:
                c.wait()
            pltpu.sync_copy(buf, out.at[pl.ds(base, BPW)])

    return k(ju2, ji2, pug, pig, pum, pim)


def _select(block, res, k0):
    acc = jnp.zeros((block.shape[0], D), jnp.float32)
    for a in range(PACK):
        acc = acc + jnp.where(res == a, block[:, a * D:(a + 1) * D], 0.0)
    del k0
    return acc


def _dense_body(gug_ref, gig_ref, gum_ref, gim_ref, ru_ref, ri_ref,
                w0u, w0i, b0, w1, b1, w2, b2, w3, b3, wpg, wph, bp,
                out_ref):
    f32 = jnp.float32
    hi = jax.lax.Precision.HIGHEST
    ru = ru_ref[...]
    ri = ri_ref[...]
    ug = _select(gug_ref[...], ru, 0)
    ig = _select(gig_ref[...], ri, 0)
    um = _select(gum_ref[...], ru, 0)
    im = _select(gim_ref[...], ri, 0)
    nu = jnp.sqrt(jnp.sum(ug * ug, axis=1, keepdims=True))
    ni = jnp.sqrt(jnp.sum(ig * ig, axis=1, keepdims=True))
    gmf = (ug / jnp.maximum(nu, 1e-12)) * (ig / jnp.maximum(ni, 1e-12))
    h = (jnp.dot(um, w0u[...], preferred_element_type=f32, precision=hi)
         + jnp.dot(im, w0i[...], preferred_element_type=f32, precision=hi)
         + b0[...])
    h = jnp.maximum(h, 0.0)
    for w, b in ((w1, b1), (w2, b2), (w3, b3)):
        h = jnp.dot(h, w[...], preferred_element_type=f32, precision=hi) + b[...]
        h = jnp.maximum(h, 0.0)
    pred = (jnp.dot(gmf, wpg[...], preferred_element_type=f32, precision=hi)
            + jnp.dot(h, wph[...], preferred_element_type=f32, precision=hi)
            + bp[...])
    out_ref[...] = jax.nn.sigmoid(pred)


def kernel(user_indices, item_indices, user_emb_gmf, item_emb_gmf,
           user_emb_mlp, item_emb_mlp,
           W0, b0, gamma0, beta0, W1, b1, gamma1, beta1,
           W2, b2, gamma2, beta2, W3, b3, gamma3, beta3,
           Wp, bp):
    uidx = user_indices.astype(jnp.int32)
    iidx = item_indices.astype(jnp.int32)
    ju = 512 * (uidx // 2048) + uidx % 512
    ji = 512 * (iidx // 2048) + iidx % 512
    ju2 = ju.reshape(BATCH // CH, CH)
    ji2 = ji.reshape(BATCH // CH, CH)
    ru = ((uidx // 512) % PACK).astype(jnp.int32).reshape(BATCH, 1)
    ri = ((iidx // 512) % PACK).astype(jnp.int32).reshape(BATCH, 1)

    packed = _tc_repack((user_emb_gmf.T, item_emb_gmf.T,
                         user_emb_mlp.T, item_emb_mlp.T))
    gug, gig, gum, gim = _sc_gather(ju2, ji2, *packed)

    bn = 1.0 / jnp.sqrt(1.0 + BN_EPS)
    def fold(W, b, g, be):
        s = g * bn
        return W * s[None, :], (b * s + be)[None, :]
    W0f, b0f = fold(W0, b0, gamma0, beta0)
    W1f, b1f = fold(W1, b1, gamma1, beta1)
    W2f, b2f = fold(W2, b2, gamma2, beta2)
    W3f, b3f = fold(W3, b3, gamma3, beta3)
    w0u, w0i = W0f[:D], W0f[D:]
    wpg, wph = Wp[:D], Wp[D:]
    bp2 = bp[None, :]

    BB = 2048
    grid = (BATCH // BB,)
    g_spec = pl.BlockSpec((BB, PACK * D), lambda i: (i, 0))
    r_spec = pl.BlockSpec((BB, 1), lambda i: (i, 0))
    def w_spec(a):
        return pl.BlockSpec(a.shape, lambda i: (0,) * a.ndim)
    weights = (w0u, w0i, b0f, W1f, b1f, W2f, b2f, W3f, b3f, wpg, wph, bp2)

    out = pl.pallas_call(
        _dense_body,
        grid=grid,
        in_specs=[g_spec] * 4 + [r_spec] * 2 + [w_spec(a) for a in weights],
        out_specs=pl.BlockSpec((BB, 1), lambda i: (i, 0)),
        out_shape=jax.ShapeDtypeStruct((BATCH, 1), jnp.float32),
    )(gug, gig, gum, gim, ru, ri, *weights)
    return out


# speedup vs baseline: 2.9496x; 1.0600x over previous
import functools

import jax
import jax.numpy as jnp
from jax import lax
from jax.experimental import pallas as pl
from jax.experimental.pallas import tpu as pltpu
from jax.experimental.pallas import tpu_sc as plsc

BATCH = 16384
D = 32
NROWS = 1000000
PACK = 4
BN_EPS = 1e-5

NC = 2
NS = 16
NW = NC * NS
BPW = BATCH // NW
CH = 128
NCH = BPW // CH

RB = 512
UB = RB * PACK
RGRID = -(-NROWS // UB)
PROWS = RGRID * RB


def _repack_body(*refs):
    ins, outs = refs[:4], refs[4:]
    base = jnp.eye(D, dtype=jnp.bfloat16)
    eyes = [jnp.pad(base, ((0, 0), (D * a, D * (PACK - 1 - a))))
            for a in range(PACK)]
    limit = NROWS - pl.program_id(0) * UB
    ok = jax.lax.broadcasted_iota(jnp.int32, (D, UB), 1) < limit
    for t in range(4):
        x = ins[t][...]
        xb = jnp.where(ok, x, 0.0).astype(jnp.bfloat16)
        acc = None
        for a in range(PACK):
            xa = xb[:, 512 * a:512 * (a + 1)]
            p = jax.lax.dot_general(xa, eyes[a], (((0,), (0,)), ((), ())),
                                    preferred_element_type=jnp.float32)
            acc = p if acc is None else acc + p
        outs[t][...] = acc


def _tc_repack(tablesT):
    in_specs = [pl.BlockSpec((D, UB), lambda i: (0, i)) for _ in range(4)]
    out_specs = [pl.BlockSpec((RB, PACK * D), lambda i: (i, 0))] * 4
    out_shape = [jax.ShapeDtypeStruct((PROWS, PACK * D), jnp.float32)] * 4
    return pl.pallas_call(
        _repack_body,
        grid=(RGRID,),
        in_specs=in_specs,
        out_specs=out_specs,
        out_shape=out_shape,
    )(*tablesT)


def _sc_gather(ju2, ji2, pug, pig, pum, pim):
    mesh = plsc.VectorSubcoreMesh(core_axis_name="c", subcore_axis_name="s")

    @functools.partial(
        pl.kernel,
        mesh=mesh,
        out_type=[jax.ShapeDtypeStruct((BATCH, PACK * D), jnp.float32)] * 4,
        scratch_types=[
            pltpu.VMEM((NCH, CH), jnp.int32),
            pltpu.VMEM((NCH, CH), jnp.int32),
            pltpu.VMEM((BPW, PACK * D), jnp.float32),
            pltpu.SemaphoreType.DMA,
        ],
    )
    def k(ju_hbm, ji_hbm, t0, t1, t2, t3, o0, o1, o2, o3, uix, iix, buf, sem):
        wid = lax.axis_index("s") * NC + lax.axis_index("c")
        base = wid * BPW
        row0 = wid * NCH
        pltpu.sync_copy(ju_hbm.at[pl.ds(row0, NCH)], uix)
        pltpu.sync_copy(ji_hbm.at[pl.ds(row0, NCH)], iix)
        for tbl, out, idx in ((t0, o0, uix), (t1, o1, iix),
                              (t2, o2, uix), (t3, o3, iix)):
            copies = []
            for j in range(NCH):
                copies.append(pltpu.async_copy(
                    tbl.at[idx.at[j]], buf.at[pl.ds(j * CH, CH)], sem))
            for c in copies:
                c.wait()
            pltpu.sync_copy(buf, out.at[pl.ds(base, BPW)])

    return k(ju2, ji2, pug, pig, pum, pim)


def _select(block, res, k0):
    acc = jnp.zeros((block.shape[0], D), jnp.float32)
    for a in range(PACK):
        acc = acc + jnp.where(res == a, block[:, a * D:(a + 1) * D], 0.0)
    del k0
    return acc


def _dense_body(gug_ref, gig_ref, gum_ref, gim_ref, ru_ref, ri_ref,
                w0u, w0i, b0, w1, b1, w2, b2, w3, b3, wpg, wph, bp,
                out_ref):
    f32 = jnp.float32
    hi = jax.lax.Precision.HIGHEST
    ru = ru_ref[...]
    ri = ri_ref[...]
    ug = _select(gug_ref[...], ru, 0)
    ig = _select(gig_ref[...], ri, 0)
    um = _select(gum_ref[...], ru, 0)
    im = _select(gim_ref[...], ri, 0)
    nu = jnp.sqrt(jnp.sum(ug * ug, axis=1, keepdims=True))
    ni = jnp.sqrt(jnp.sum(ig * ig, axis=1, keepdims=True))
    gmf = (ug / jnp.maximum(nu, 1e-12)) * (ig / jnp.maximum(ni, 1e-12))
    h = (jnp.dot(um, w0u[...], preferred_element_type=f32)
         + jnp.dot(im, w0i[...], preferred_element_type=f32)
         + b0[...])
    h = jnp.maximum(h, 0.0)
    for w, b in ((w1, b1), (w2, b2), (w3, b3)):
        h = jnp.dot(h, w[...], preferred_element_type=f32) + b[...]
        h = jnp.maximum(h, 0.0)
    pred = (jnp.dot(gmf, wpg[...], preferred_element_type=f32)
            + jnp.dot(h, wph[...], preferred_element_type=f32)
            + bp[...])
    out_ref[...] = jax.nn.sigmoid(pred)


def kernel(user_indices, item_indices, user_emb_gmf, item_emb_gmf,
           user_emb_mlp, item_emb_mlp,
           W0, b0, gamma0, beta0, W1, b1, gamma1, beta1,
           W2, b2, gamma2, beta2, W3, b3, gamma3, beta3,
           Wp, bp):
    uidx = user_indices.astype(jnp.int32)
    iidx = item_indices.astype(jnp.int32)
    ju = 512 * (uidx // 2048) + uidx % 512
    ji = 512 * (iidx // 2048) + iidx % 512
    ju2 = ju.reshape(BATCH // CH, CH)
    ji2 = ji.reshape(BATCH // CH, CH)
    ru = ((uidx // 512) % PACK).astype(jnp.int32).reshape(BATCH, 1)
    ri = ((iidx // 512) % PACK).astype(jnp.int32).reshape(BATCH, 1)

    packed = _tc_repack((user_emb_gmf.T, item_emb_gmf.T,
                         user_emb_mlp.T, item_emb_mlp.T))
    gug, gig, gum, gim = _sc_gather(ju2, ji2, *packed)

    bn = 1.0 / jnp.sqrt(1.0 + BN_EPS)
    def fold(W, b, g, be):
        s = g * bn
        return W * s[None, :], (b * s + be)[None, :]
    W0f, b0f = fold(W0, b0, gamma0, beta0)
    W1f, b1f = fold(W1, b1, gamma1, beta1)
    W2f, b2f = fold(W2, b2, gamma2, beta2)
    W3f, b3f = fold(W3, b3, gamma3, beta3)
    w0u, w0i = W0f[:D], W0f[D:]
    wpg, wph = Wp[:D], Wp[D:]
    bp2 = bp[None, :]

    BB = 2048
    grid = (BATCH // BB,)
    g_spec = pl.BlockSpec((BB, PACK * D), lambda i: (i, 0))
    r_spec = pl.BlockSpec((BB, 1), lambda i: (i, 0))
    def w_spec(a):
        return pl.BlockSpec(a.shape, lambda i: (0,) * a.ndim)
    weights = (w0u, w0i, b0f, W1f, b1f, W2f, b2f, W3f, b3f, wpg, wph, bp2)

    out = pl.pallas_call(
        _dense_body,
        grid=grid,
        in_specs=[g_spec] * 4 + [r_spec] * 2 + [w_spec(a) for a in weights],
        out_specs=pl.BlockSpec((BB, 1), lambda i: (i, 0)),
        out_shape=jax.ShapeDtypeStruct((BATCH, 1), jnp.float32),
    )(gug, gig, gum, gim, ru, ri, *weights)
    return out


# speedup vs baseline: 3.6989x; 1.2540x over previous
import functools

import jax
import jax.numpy as jnp
from jax import lax
from jax.experimental import pallas as pl
from jax.experimental.pallas import tpu as pltpu
from jax.experimental.pallas import tpu_sc as plsc

BATCH = 16384
D = 32
NROWS = 1000000
PACK = 4
BN_EPS = 1e-5

NC = 2
NS = 16
NW = NC * NS
BPW = BATCH // NW
CH = 128
NCH = BPW // CH

RB = 1024
UB = RB * PACK
RGRID = -(-NROWS // UB)
PROWS = RGRID * RB


def _repack_body(*refs):
    ins, outs = refs[:4], refs[4:]
    base = jnp.eye(D, dtype=jnp.bfloat16)
    eyes = [jnp.pad(base, ((0, 0), (D * a, D * (PACK - 1 - a))))
            for a in range(PACK)]
    limit = NROWS - pl.program_id(0) * UB
    ok = jax.lax.broadcasted_iota(jnp.int32, (D, UB), 1) < limit
    for t in range(4):
        x = ins[t][...]
        xb = jnp.where(ok, x, 0.0).astype(jnp.bfloat16)
        acc = None
        for a in range(PACK):
            xa = xb[:, RB * a:RB * (a + 1)]
            p = jax.lax.dot_general(xa, eyes[a], (((0,), (0,)), ((), ())),
                                    preferred_element_type=jnp.float32)
            acc = p if acc is None else acc + p
        outs[t][...] = acc


def _tc_repack(tablesT):
    in_specs = [pl.BlockSpec((D, UB), lambda i: (0, i)) for _ in range(4)]
    out_specs = [pl.BlockSpec((RB, PACK * D), lambda i: (i, 0))] * 4
    out_shape = [jax.ShapeDtypeStruct((PROWS, PACK * D), jnp.float32)] * 4
    return pl.pallas_call(
        _repack_body,
        grid=(RGRID,),
        in_specs=in_specs,
        out_specs=out_specs,
        out_shape=out_shape,
    )(*tablesT)


def _sc_gather(ju2, ji2, pug, pig, pum, pim):
    mesh = plsc.VectorSubcoreMesh(core_axis_name="c", subcore_axis_name="s")

    @functools.partial(
        pl.kernel,
        mesh=mesh,
        out_type=[jax.ShapeDtypeStruct((BATCH, PACK * D), jnp.float32)] * 4,
        scratch_types=[
            pltpu.VMEM((NCH, CH), jnp.int32),
            pltpu.VMEM((NCH, CH), jnp.int32),
            pltpu.VMEM((BPW, PACK * D), jnp.float32),
            pltpu.SemaphoreType.DMA,
        ],
    )
    def k(ju_hbm, ji_hbm, t0, t1, t2, t3, o0, o1, o2, o3, uix, iix, buf, sem):
        wid = lax.axis_index("s") * NC + lax.axis_index("c")
        base = wid * BPW
        row0 = wid * NCH
        pltpu.sync_copy(ju_hbm.at[pl.ds(row0, NCH)], uix)
        pltpu.sync_copy(ji_hbm.at[pl.ds(row0, NCH)], iix)
        for tbl, out, idx in ((t0, o0, uix), (t1, o1, iix),
                              (t2, o2, uix), (t3, o3, iix)):
            copies = []
            for j in range(NCH):
                copies.append(pltpu.async_copy(
                    tbl.at[idx.at[j]], buf.at[pl.ds(j * CH, CH)], sem))
            for c in copies:
                c.wait()
            pltpu.sync_copy(buf, out.at[pl.ds(base, BPW)])

    return k(ju2, ji2, pug, pig, pum, pim)


def _select(block, res, k0):
    acc = jnp.zeros((block.shape[0], D), jnp.float32)
    for a in range(PACK):
        acc = acc + jnp.where(res == a, block[:, a * D:(a + 1) * D], 0.0)
    del k0
    return acc


def _dense_body(gug_ref, gig_ref, gum_ref, gim_ref, ru_ref, ri_ref,
                w0u, w0i, b0, w1, b1, w2, b2, w3, b3, wpg, wph, bp,
                out_ref):
    f32 = jnp.float32
    hi = jax.lax.Precision.HIGHEST
    ru = ru_ref[...]
    ri = ri_ref[...]
    ug = _select(gug_ref[...], ru, 0)
    ig = _select(gig_ref[...], ri, 0)
    um = _select(gum_ref[...], ru, 0)
    im = _select(gim_ref[...], ri, 0)
    nu = jnp.sqrt(jnp.sum(ug * ug, axis=1, keepdims=True))
    ni = jnp.sqrt(jnp.sum(ig * ig, axis=1, keepdims=True))
    gmf = (ug / jnp.maximum(nu, 1e-12)) * (ig / jnp.maximum(ni, 1e-12))
    h = (jnp.dot(um, w0u[...], preferred_element_type=f32)
         + jnp.dot(im, w0i[...], preferred_element_type=f32)
         + b0[...])
    h = jnp.maximum(h, 0.0)
    for w, b in ((w1, b1), (w2, b2), (w3, b3)):
        h = jnp.dot(h, w[...], preferred_element_type=f32) + b[...]
        h = jnp.maximum(h, 0.0)
    pred = (jnp.dot(gmf, wpg[...], preferred_element_type=f32)
            + jnp.dot(h, wph[...], preferred_element_type=f32)
            + bp[...])
    out_ref[...] = jax.nn.sigmoid(pred)


def kernel(user_indices, item_indices, user_emb_gmf, item_emb_gmf,
           user_emb_mlp, item_emb_mlp,
           W0, b0, gamma0, beta0, W1, b1, gamma1, beta1,
           W2, b2, gamma2, beta2, W3, b3, gamma3, beta3,
           Wp, bp):
    uidx = user_indices.astype(jnp.int32)
    iidx = item_indices.astype(jnp.int32)
    ju = RB * (uidx // UB) + uidx % RB
    ji = RB * (iidx // UB) + iidx % RB
    ju2 = ju.reshape(BATCH // CH, CH)
    ji2 = ji.reshape(BATCH // CH, CH)
    ru = ((uidx // RB) % PACK).astype(jnp.int32).reshape(BATCH, 1)
    ri = ((iidx // RB) % PACK).astype(jnp.int32).reshape(BATCH, 1)

    packed = _tc_repack((user_emb_gmf.T, item_emb_gmf.T,
                         user_emb_mlp.T, item_emb_mlp.T))
    gug, gig, gum, gim = _sc_gather(ju2, ji2, *packed)

    bn = 1.0 / jnp.sqrt(1.0 + BN_EPS)
    def fold(W, b, g, be):
        s = g * bn
        return W * s[None, :], (b * s + be)[None, :]
    W0f, b0f = fold(W0, b0, gamma0, beta0)
    W1f, b1f = fold(W1, b1, gamma1, beta1)
    W2f, b2f = fold(W2, b2, gamma2, beta2)
    W3f, b3f = fold(W3, b3, gamma3, beta3)
    w0u, w0i = W0f[:D], W0f[D:]
    wpg, wph = Wp[:D], Wp[D:]
    bp2 = bp[None, :]

    BB = 2048
    grid = (BATCH // BB,)
    g_spec = pl.BlockSpec((BB, PACK * D), lambda i: (i, 0))
    r_spec = pl.BlockSpec((BB, 1), lambda i: (i, 0))
    def w_spec(a):
        return pl.BlockSpec(a.shape, lambda i: (0,) * a.ndim)
    weights = (w0u, w0i, b0f, W1f, b1f, W2f, b2f, W3f, b3f, wpg, wph, bp2)

    out = pl.pallas_call(
        _dense_body,
        grid=grid,
        in_specs=[g_spec] * 4 + [r_spec] * 2 + [w_spec(a) for a in weights],
        out_specs=pl.BlockSpec((BB, 1), lambda i: (i, 0)),
        out_shape=jax.ShapeDtypeStruct((BATCH, 1), jnp.float32),
    )(gug, gig, gum, gim, ru, ri, *weights)
    return out


# speedup vs baseline: 4.3073x; 1.1645x over previous
import functools

import jax
import jax.numpy as jnp
from jax import lax
from jax.experimental import pallas as pl
from jax.experimental.pallas import tpu as pltpu
from jax.experimental.pallas import tpu_sc as plsc

BATCH = 16384
D = 32
NROWS = 1000000
PACK = 4
BN_EPS = 1e-5

NC = 2
NS = 16
NW = NC * NS
BPW = BATCH // NW
CH = 128
NCH = BPW // CH

RB = 2048
UB = RB * PACK
RGRID = -(-NROWS // UB)
PROWS = RGRID * RB


def _repack_body(*refs):
    ins, outs = refs[:4], refs[4:]
    base = jnp.eye(D, dtype=jnp.bfloat16)
    eyes = [jnp.pad(base, ((0, 0), (D * a, D * (PACK - 1 - a))))
            for a in range(PACK)]
    limit = NROWS - pl.program_id(0) * UB
    ok = jax.lax.broadcasted_iota(jnp.int32, (D, UB), 1) < limit
    for t in range(4):
        x = ins[t][...]
        xb = jnp.where(ok, x, 0.0).astype(jnp.bfloat16)
        acc = None
        for a in range(PACK):
            xa = xb[:, RB * a:RB * (a + 1)]
            p = jax.lax.dot_general(xa, eyes[a], (((0,), (0,)), ((), ())),
                                    preferred_element_type=jnp.float32)
            acc = p if acc is None else acc + p
        outs[t][...] = acc


def _tc_repack(tablesT):
    in_specs = [pl.BlockSpec((D, UB), lambda i: (0, i)) for _ in range(4)]
    out_specs = [pl.BlockSpec((RB, PACK * D), lambda i: (i, 0))] * 4
    out_shape = [jax.ShapeDtypeStruct((PROWS, PACK * D), jnp.float32)] * 4
    return pl.pallas_call(
        _repack_body,
        grid=(RGRID,),
        in_specs=in_specs,
        out_specs=out_specs,
        out_shape=out_shape,
    )(*tablesT)


def _sc_gather(ju2, ji2, pug, pig, pum, pim):
    mesh = plsc.VectorSubcoreMesh(core_axis_name="c", subcore_axis_name="s")

    @functools.partial(
        pl.kernel,
        mesh=mesh,
        out_type=[jax.ShapeDtypeStruct((BATCH, PACK * D), jnp.float32)] * 4,
        scratch_types=[
            pltpu.VMEM((NCH, CH), jnp.int32),
            pltpu.VMEM((NCH, CH), jnp.int32),
            pltpu.VMEM((BPW, PACK * D), jnp.float32),
            pltpu.SemaphoreType.DMA,
        ],
    )
    def k(ju_hbm, ji_hbm, t0, t1, t2, t3, o0, o1, o2, o3, uix, iix, buf, sem):
        wid = lax.axis_index("s") * NC + lax.axis_index("c")
        base = wid * BPW
        row0 = wid * NCH
        pltpu.sync_copy(ju_hbm.at[pl.ds(row0, NCH)], uix)
        pltpu.sync_copy(ji_hbm.at[pl.ds(row0, NCH)], iix)
        for tbl, out, idx in ((t0, o0, uix), (t1, o1, iix),
                              (t2, o2, uix), (t3, o3, iix)):
            copies = []
            for j in range(NCH):
                copies.append(pltpu.async_copy(
                    tbl.at[idx.at[j]], buf.at[pl.ds(j * CH, CH)], sem))
            for c in copies:
                c.wait()
            pltpu.sync_copy(buf, out.at[pl.ds(base, BPW)])

    return k(ju2, ji2, pug, pig, pum, pim)


def _select(block, res, k0):
    acc = jnp.zeros((block.shape[0], D), jnp.float32)
    for a in range(PACK):
        acc = acc + jnp.where(res == a, block[:, a * D:(a + 1) * D], 0.0)
    del k0
    return acc


def _dense_body(gug_ref, gig_ref, gum_ref, gim_ref, ru_ref, ri_ref,
                w0u, w0i, b0, w1, b1, w2, b2, w3, b3, wpg, wph, bp,
                out_ref):
    f32 = jnp.float32
    hi = jax.lax.Precision.HIGHEST
    ru = ru_ref[...]
    ri = ri_ref[...]
    ug = _select(gug_ref[...], ru, 0)
    ig = _select(gig_ref[...], ri, 0)
    um = _select(gum_ref[...], ru, 0)
    im = _select(gim_ref[...], ri, 0)
    nu = jnp.sqrt(jnp.sum(ug * ug, axis=1, keepdims=True))
    ni = jnp.sqrt(jnp.sum(ig * ig, axis=1, keepdims=True))
    gmf = (ug / jnp.maximum(nu, 1e-12)) * (ig / jnp.maximum(ni, 1e-12))
    h = (jnp.dot(um, w0u[...], preferred_element_type=f32)
         + jnp.dot(im, w0i[...], preferred_element_type=f32)
         + b0[...])
    h = jnp.maximum(h, 0.0)
    for w, b in ((w1, b1), (w2, b2), (w3, b3)):
        h = jnp.dot(h, w[...], preferred_element_type=f32) + b[...]
        h = jnp.maximum(h, 0.0)
    pred = (jnp.dot(gmf, wpg[...], preferred_element_type=f32)
            + jnp.dot(h, wph[...], preferred_element_type=f32)
            + bp[...])
    out_ref[...] = jax.nn.sigmoid(pred)


def kernel(user_indices, item_indices, user_emb_gmf, item_emb_gmf,
           user_emb_mlp, item_emb_mlp,
           W0, b0, gamma0, beta0, W1, b1, gamma1, beta1,
           W2, b2, gamma2, beta2, W3, b3, gamma3, beta3,
           Wp, bp):
    uidx = user_indices.astype(jnp.int32)
    iidx = item_indices.astype(jnp.int32)
    ju = RB * (uidx // UB) + uidx % RB
    ji = RB * (iidx // UB) + iidx % RB
    ju2 = ju.reshape(BATCH // CH, CH)
    ji2 = ji.reshape(BATCH // CH, CH)
    ru = ((uidx // RB) % PACK).astype(jnp.int32).reshape(BATCH, 1)
    ri = ((iidx // RB) % PACK).astype(jnp.int32).reshape(BATCH, 1)

    packed = _tc_repack((user_emb_gmf.T, item_emb_gmf.T,
                         user_emb_mlp.T, item_emb_mlp.T))
    gug, gig, gum, gim = _sc_gather(ju2, ji2, *packed)

    bn = 1.0 / jnp.sqrt(1.0 + BN_EPS)
    def fold(W, b, g, be):
        s = g * bn
        return W * s[None, :], (b * s + be)[None, :]
    W0f, b0f = fold(W0, b0, gamma0, beta0)
    W1f, b1f = fold(W1, b1, gamma1, beta1)
    W2f, b2f = fold(W2, b2, gamma2, beta2)
    W3f, b3f = fold(W3, b3, gamma3, beta3)
    w0u, w0i = W0f[:D], W0f[D:]
    wpg, wph = Wp[:D], Wp[D:]
    bp2 = bp[None, :]

    BB = 2048
    grid = (BATCH // BB,)
    g_spec = pl.BlockSpec((BB, PACK * D), lambda i: (i, 0))
    r_spec = pl.BlockSpec((BB, 1), lambda i: (i, 0))
    def w_spec(a):
        return pl.BlockSpec(a.shape, lambda i: (0,) * a.ndim)
    weights = (w0u, w0i, b0f, W1f, b1f, W2f, b2f, W3f, b3f, wpg, wph, bp2)

    out = pl.pallas_call(
        _dense_body,
        grid=grid,
        in_specs=[g_spec] * 4 + [r_spec] * 2 + [w_spec(a) for a in weights],
        out_specs=pl.BlockSpec((BB, 1), lambda i: (i, 0)),
        out_shape=jax.ShapeDtypeStruct((BATCH, 1), jnp.float32),
    )(gug, gig, gum, gim, ru, ri, *weights)
    return out


# speedup vs baseline: 4.7058x; 1.0925x over previous
import functools

import jax
import jax.numpy as jnp
from jax import lax
from jax.experimental import pallas as pl
from jax.experimental.pallas import tpu as pltpu
from jax.experimental.pallas import tpu_sc as plsc

BATCH = 16384
D = 32
NROWS = 1000000
PACK = 4
BN_EPS = 1e-5

NC = 2
NS = 16
NW = NC * NS
BPW = BATCH // NW
CH = 128
NCH = BPW // CH

RB = 4096
UB = RB * PACK
RGRID = -(-NROWS // UB)
PROWS = RGRID * RB


def _repack_body(*refs):
    ins, outs = refs[:4], refs[4:]
    base = jnp.eye(D, dtype=jnp.bfloat16)
    eyes = [jnp.pad(base, ((0, 0), (D * a, D * (PACK - 1 - a))))
            for a in range(PACK)]
    limit = NROWS - pl.program_id(0) * UB
    ok = jax.lax.broadcasted_iota(jnp.int32, (D, UB), 1) < limit
    for t in range(4):
        x = ins[t][...]
        xb = jnp.where(ok, x, 0.0).astype(jnp.bfloat16)
        acc = None
        for a in range(PACK):
            xa = xb[:, RB * a:RB * (a + 1)]
            p = jax.lax.dot_general(xa, eyes[a], (((0,), (0,)), ((), ())),
                                    preferred_element_type=jnp.float32)
            acc = p if acc is None else acc + p
        outs[t][...] = acc


def _tc_repack(tablesT):
    in_specs = [pl.BlockSpec((D, UB), lambda i: (0, i)) for _ in range(4)]
    out_specs = [pl.BlockSpec((RB, PACK * D), lambda i: (i, 0))] * 4
    out_shape = [jax.ShapeDtypeStruct((PROWS, PACK * D), jnp.float32)] * 4
    return pl.pallas_call(
        _repack_body,
        grid=(RGRID,),
        in_specs=in_specs,
        out_specs=out_specs,
        out_shape=out_shape,
    )(*tablesT)


def _sc_gather(ju2, ji2, pug, pig, pum, pim):
    mesh = plsc.VectorSubcoreMesh(core_axis_name="c", subcore_axis_name="s")

    @functools.partial(
        pl.kernel,
        mesh=mesh,
        out_type=[jax.ShapeDtypeStruct((BATCH, PACK * D), jnp.float32)] * 4,
        scratch_types=[
            pltpu.VMEM((NCH, CH), jnp.int32),
            pltpu.VMEM((NCH, CH), jnp.int32),
            pltpu.VMEM((BPW, PACK * D), jnp.float32),
            pltpu.SemaphoreType.DMA,
        ],
    )
    def k(ju_hbm, ji_hbm, t0, t1, t2, t3, o0, o1, o2, o3, uix, iix, buf, sem):
        wid = lax.axis_index("s") * NC + lax.axis_index("c")
        base = wid * BPW
        row0 = wid * NCH
        pltpu.sync_copy(ju_hbm.at[pl.ds(row0, NCH)], uix)
        pltpu.sync_copy(ji_hbm.at[pl.ds(row0, NCH)], iix)
        for tbl, out, idx in ((t0, o0, uix), (t1, o1, iix),
                              (t2, o2, uix), (t3, o3, iix)):
            copies = []
            for j in range(NCH):
                copies.append(pltpu.async_copy(
                    tbl.at[idx.at[j]], buf.at[pl.ds(j * CH, CH)], sem))
            for c in copies:
                c.wait()
            pltpu.sync_copy(buf, out.at[pl.ds(base, BPW)])

    return k(ju2, ji2, pug, pig, pum, pim)


def _select(block, res, k0):
    acc = jnp.zeros((block.shape[0], D), jnp.float32)
    for a in range(PACK):
        acc = acc + jnp.where(res == a, block[:, a * D:(a + 1) * D], 0.0)
    del k0
    return acc


def _dense_body(gug_ref, gig_ref, gum_ref, gim_ref, ru_ref, ri_ref,
                w0u, w0i, b0, w1, b1, w2, b2, w3, b3, wpg, wph, bp,
                out_ref):
    f32 = jnp.float32
    hi = jax.lax.Precision.HIGHEST
    ru = ru_ref[...]
    ri = ri_ref[...]
    ug = _select(gug_ref[...], ru, 0)
    ig = _select(gig_ref[...], ri, 0)
    um = _select(gum_ref[...], ru, 0)
    im = _select(gim_ref[...], ri, 0)
    nu = jnp.sqrt(jnp.sum(ug * ug, axis=1, keepdims=True))
    ni = jnp.sqrt(jnp.sum(ig * ig, axis=1, keepdims=True))
    gmf = (ug / jnp.maximum(nu, 1e-12)) * (ig / jnp.maximum(ni, 1e-12))
    h = (jnp.dot(um, w0u[...], preferred_element_type=f32)
         + jnp.dot(im, w0i[...], preferred_element_type=f32)
         + b0[...])
    h = jnp.maximum(h, 0.0)
    for w, b in ((w1, b1), (w2, b2), (w3, b3)):
        h = jnp.dot(h, w[...], preferred_element_type=f32) + b[...]
        h = jnp.maximum(h, 0.0)
    pred = (jnp.dot(gmf, wpg[...], preferred_element_type=f32)
            + jnp.dot(h, wph[...], preferred_element_type=f32)
            + bp[...])
    out_ref[...] = jax.nn.sigmoid(pred)


def kernel(user_indices, item_indices, user_emb_gmf, item_emb_gmf,
           user_emb_mlp, item_emb_mlp,
           W0, b0, gamma0, beta0, W1, b1, gamma1, beta1,
           W2, b2, gamma2, beta2, W3, b3, gamma3, beta3,
           Wp, bp):
    uidx = user_indices.astype(jnp.int32)
    iidx = item_indices.astype(jnp.int32)
    ju = RB * (uidx // UB) + uidx % RB
    ji = RB * (iidx // UB) + iidx % RB
    ju2 = ju.reshape(BATCH // CH, CH)
    ji2 = ji.reshape(BATCH // CH, CH)
    ru = ((uidx // RB) % PACK).astype(jnp.int32).reshape(BATCH, 1)
    ri = ((iidx // RB) % PACK).astype(jnp.int32).reshape(BATCH, 1)

    packed = _tc_repack((user_emb_gmf.T, item_emb_gmf.T,
                         user_emb_mlp.T, item_emb_mlp.T))
    gug, gig, gum, gim = _sc_gather(ju2, ji2, *packed)

    bn = 1.0 / jnp.sqrt(1.0 + BN_EPS)
    def fold(W, b, g, be):
        s = g * bn
        return W * s[None, :], (b * s + be)[None, :]
    W0f, b0f = fold(W0, b0, gamma0, beta0)
    W1f, b1f = fold(W1, b1, gamma1, beta1)
    W2f, b2f = fold(W2, b2, gamma2, beta2)
    W3f, b3f = fold(W3, b3, gamma3, beta3)
    w0u, w0i = W0f[:D], W0f[D:]
    wpg, wph = Wp[:D], Wp[D:]
    bp2 = bp[None, :]

    BB = 2048
    grid = (BATCH // BB,)
    g_spec = pl.BlockSpec((BB, PACK * D), lambda i: (i, 0))
    r_spec = pl.BlockSpec((BB, 1), lambda i: (i, 0))
    def w_spec(a):
        return pl.BlockSpec(a.shape, lambda i: (0,) * a.ndim)
    weights = (w0u, w0i, b0f, W1f, b1f, W2f, b2f, W3f, b3f, wpg, wph, bp2)

    out = pl.pallas_call(
        _dense_body,
        grid=grid,
        in_specs=[g_spec] * 4 + [r_spec] * 2 + [w_spec(a) for a in weights],
        out_specs=pl.BlockSpec((BB, 1), lambda i: (i, 0)),
        out_shape=jax.ShapeDtypeStruct((BATCH, 1), jnp.float32),
    )(gug, gig, gum, gim, ru, ri, *weights)
    return out


# speedup vs baseline: 4.8402x; 1.0286x over previous
import functools

import jax
import jax.numpy as jnp
from jax import lax
from jax.experimental import pallas as pl
from jax.experimental.pallas import tpu as pltpu
from jax.experimental.pallas import tpu_sc as plsc

BATCH = 16384
D = 32
NROWS = 1000000
PACK = 4
BN_EPS = 1e-5

NC = 2
NS = 16
NW = NC * NS
BPW = BATCH // NW
CH = 128
NCH = BPW // CH

RB = 6144
UB = RB * PACK
RGRID = -(-NROWS // UB)
PROWS = RGRID * RB


def _repack_body(*refs):
    ins, outs = refs[:4], refs[4:]
    base = jnp.eye(D, dtype=jnp.bfloat16)
    eyes = [jnp.pad(base, ((0, 0), (D * a, D * (PACK - 1 - a))))
            for a in range(PACK)]
    limit = NROWS - pl.program_id(0) * UB
    ok = jax.lax.broadcasted_iota(jnp.int32, (D, UB), 1) < limit
    for t in range(4):
        x = ins[t][...]
        xb = jnp.where(ok, x, 0.0).astype(jnp.bfloat16)
        acc = None
        for a in range(PACK):
            xa = xb[:, RB * a:RB * (a + 1)]
            p = jax.lax.dot_general(xa, eyes[a], (((0,), (0,)), ((), ())),
                                    preferred_element_type=jnp.float32)
            acc = p if acc is None else acc + p
        outs[t][...] = acc


def _tc_repack(tablesT):
    in_specs = [pl.BlockSpec((D, UB), lambda i: (0, i)) for _ in range(4)]
    out_specs = [pl.BlockSpec((RB, PACK * D), lambda i: (i, 0))] * 4
    out_shape = [jax.ShapeDtypeStruct((PROWS, PACK * D), jnp.float32)] * 4
    return pl.pallas_call(
        _repack_body,
        grid=(RGRID,),
        in_specs=in_specs,
        out_specs=out_specs,
        out_shape=out_shape,
    )(*tablesT)


def _sc_gather(ju2, ji2, pug, pig, pum, pim):
    mesh = plsc.VectorSubcoreMesh(core_axis_name="c", subcore_axis_name="s")

    @functools.partial(
        pl.kernel,
        mesh=mesh,
        out_type=[jax.ShapeDtypeStruct((BATCH, PACK * D), jnp.float32)] * 4,
        scratch_types=[
            pltpu.VMEM((NCH, CH), jnp.int32),
            pltpu.VMEM((NCH, CH), jnp.int32),
            pltpu.VMEM((BPW, PACK * D), jnp.float32),
            pltpu.SemaphoreType.DMA,
        ],
    )
    def k(ju_hbm, ji_hbm, t0, t1, t2, t3, o0, o1, o2, o3, uix, iix, buf, sem):
        wid = lax.axis_index("s") * NC + lax.axis_index("c")
        base = wid * BPW
        row0 = wid * NCH
        pltpu.sync_copy(ju_hbm.at[pl.ds(row0, NCH)], uix)
        pltpu.sync_copy(ji_hbm.at[pl.ds(row0, NCH)], iix)
        for tbl, out, idx in ((t0, o0, uix), (t1, o1, iix),
                              (t2, o2, uix), (t3, o3, iix)):
            copies = []
            for j in range(NCH):
                copies.append(pltpu.async_copy(
                    tbl.at[idx.at[j]], buf.at[pl.ds(j * CH, CH)], sem))
            for c in copies:
                c.wait()
            pltpu.sync_copy(buf, out.at[pl.ds(base, BPW)])

    return k(ju2, ji2, pug, pig, pum, pim)


def _select(block, res, k0):
    acc = jnp.zeros((block.shape[0], D), jnp.float32)
    for a in range(PACK):
        acc = acc + jnp.where(res == a, block[:, a * D:(a + 1) * D], 0.0)
    del k0
    return acc


def _dense_body(gug_ref, gig_ref, gum_ref, gim_ref, ru_ref, ri_ref,
                w0u, w0i, b0, w1, b1, w2, b2, w3, b3, wpg, wph, bp,
                out_ref):
    f32 = jnp.float32
    hi = jax.lax.Precision.HIGHEST
    ru = ru_ref[...]
    ri = ri_ref[...]
    ug = _select(gug_ref[...], ru, 0)
    ig = _select(gig_ref[...], ri, 0)
    um = _select(gum_ref[...], ru, 0)
    im = _select(gim_ref[...], ri, 0)
    nu = jnp.sqrt(jnp.sum(ug * ug, axis=1, keepdims=True))
    ni = jnp.sqrt(jnp.sum(ig * ig, axis=1, keepdims=True))
    gmf = (ug / jnp.maximum(nu, 1e-12)) * (ig / jnp.maximum(ni, 1e-12))
    h = (jnp.dot(um, w0u[...], preferred_element_type=f32)
         + jnp.dot(im, w0i[...], preferred_element_type=f32)
         + b0[...])
    h = jnp.maximum(h, 0.0)
    for w, b in ((w1, b1), (w2, b2), (w3, b3)):
        h = jnp.dot(h, w[...], preferred_element_type=f32) + b[...]
        h = jnp.maximum(h, 0.0)
    pred = (jnp.dot(gmf, wpg[...], preferred_element_type=f32)
            + jnp.dot(h, wph[...], preferred_element_type=f32)
            + bp[...])
    out_ref[...] = jax.nn.sigmoid(pred)


def kernel(user_indices, item_indices, user_emb_gmf, item_emb_gmf,
           user_emb_mlp, item_emb_mlp,
           W0, b0, gamma0, beta0, W1, b1, gamma1, beta1,
           W2, b2, gamma2, beta2, W3, b3, gamma3, beta3,
           Wp, bp):
    uidx = user_indices.astype(jnp.int32)
    iidx = item_indices.astype(jnp.int32)
    ju = RB * (uidx // UB) + uidx % RB
    ji = RB * (iidx // UB) + iidx % RB
    ju2 = ju.reshape(BATCH // CH, CH)
    ji2 = ji.reshape(BATCH // CH, CH)
    ru = ((uidx // RB) % PACK).astype(jnp.int32).reshape(BATCH, 1)
    ri = ((iidx // RB) % PACK).astype(jnp.int32).reshape(BATCH, 1)

    packed = _tc_repack((user_emb_gmf.T, item_emb_gmf.T,
                         user_emb_mlp.T, item_emb_mlp.T))
    gug, gig, gum, gim = _sc_gather(ju2, ji2, *packed)

    bn = 1.0 / jnp.sqrt(1.0 + BN_EPS)
    def fold(W, b, g, be):
        s = g * bn
        return W * s[None, :], (b * s + be)[None, :]
    W0f, b0f = fold(W0, b0, gamma0, beta0)
    W1f, b1f = fold(W1, b1, gamma1, beta1)
    W2f, b2f = fold(W2, b2, gamma2, beta2)
    W3f, b3f = fold(W3, b3, gamma3, beta3)
    w0u, w0i = W0f[:D], W0f[D:]
    wpg, wph = Wp[:D], Wp[D:]
    bp2 = bp[None, :]

    BB = 2048
    grid = (BATCH // BB,)
    g_spec = pl.BlockSpec((BB, PACK * D), lambda i: (i, 0))
    r_spec = pl.BlockSpec((BB, 1), lambda i: (i, 0))
    def w_spec(a):
        return pl.BlockSpec(a.shape, lambda i: (0,) * a.ndim)
    weights = (w0u, w0i, b0f, W1f, b1f, W2f, b2f, W3f, b3f, wpg, wph, bp2)

    out = pl.pallas_call(
        _dense_body,
        grid=grid,
        in_specs=[g_spec] * 4 + [r_spec] * 2 + [w_spec(a) for a in weights],
        out_specs=pl.BlockSpec((BB, 1), lambda i: (i, 0)),
        out_shape=jax.ShapeDtypeStruct((BATCH, 1), jnp.float32),
    )(gug, gig, gum, gim, ru, ri, *weights)
    return out


# speedup vs baseline: 5.7520x; 1.1884x over previous
import functools

import jax
import jax.numpy as jnp
from jax import lax
from jax.experimental import pallas as pl
from jax.experimental.pallas import tpu as pltpu
from jax.experimental.pallas import tpu_sc as plsc

BATCH = 16384
D = 32
NROWS = 1000000
PACK = 4
BN_EPS = 1e-5

NC = 2
NS = 16
NW = NC * NS
BPW = BATCH // NW
CH = 128
NCH = BPW // CH

RB = 8192
UB = RB * PACK
RGRID = -(-NROWS // UB)
PROWS = RGRID * RB


def _repack_body(ug_ref, um_ref, ig_ref, im_ref, out_u, out_i):
    base = jnp.eye(D, dtype=jnp.bfloat16)
    eyes = [jnp.pad(base, ((0, 0), (D * a, D * (PACK - 1 - a))))
            for a in range(PACK)]
    limit = NROWS - pl.program_id(0) * UB
    ok = jax.lax.broadcasted_iota(jnp.int32, (D, UB), 1) < limit

    def transpose_pack(ref):
        xb = jnp.where(ok, ref[...], 0.0).astype(jnp.bfloat16)
        acc = None
        for a in range(PACK):
            xa = xb[:, RB * a:RB * (a + 1)]
            p = jax.lax.dot_general(xa, eyes[a], (((0,), (0,)), ((), ())),
                                    preferred_element_type=jnp.float32)
            acc = p if acc is None else acc + p
        return acc

    u32 = jnp.uint32
    for hi_ref, lo_ref, out in ((ug_ref, um_ref, out_u),
                                (ig_ref, im_ref, out_i)):
        hi = jax.lax.bitcast_convert_type(transpose_pack(hi_ref), u32)
        lo = jax.lax.bitcast_convert_type(transpose_pack(lo_ref), u32)
        packed = hi | (lo >> 16)
        out[...] = jax.lax.bitcast_convert_type(packed, jnp.float32)


def _tc_repack(tablesT):
    in_specs = [pl.BlockSpec((D, UB), lambda i: (0, i)) for _ in range(4)]
    out_specs = [pl.BlockSpec((RB, PACK * D), lambda i: (i, 0))] * 2
    out_shape = [jax.ShapeDtypeStruct((PROWS, PACK * D), jnp.float32)] * 2
    return pl.pallas_call(
        _repack_body,
        grid=(RGRID,),
        in_specs=in_specs,
        out_specs=out_specs,
        out_shape=out_shape,
    )(*tablesT)


def _sc_gather(ju2, ji2, pu, pi):
    mesh = plsc.VectorSubcoreMesh(core_axis_name="c", subcore_axis_name="s")

    @functools.partial(
        pl.kernel,
        mesh=mesh,
        out_type=[jax.ShapeDtypeStruct((BATCH, PACK * D), jnp.float32)] * 2,
        scratch_types=[
            pltpu.VMEM((NCH, CH), jnp.int32),
            pltpu.VMEM((NCH, CH), jnp.int32),
            pltpu.VMEM((BPW, PACK * D), jnp.float32),
            pltpu.SemaphoreType.DMA,
        ],
    )
    def k(ju_hbm, ji_hbm, t0, t1, o0, o1, uix, iix, buf, sem):
        wid = lax.axis_index("s") * NC + lax.axis_index("c")
        base = wid * BPW
        row0 = wid * NCH
        pltpu.sync_copy(ju_hbm.at[pl.ds(row0, NCH)], uix)
        pltpu.sync_copy(ji_hbm.at[pl.ds(row0, NCH)], iix)
        for tbl, out, idx in ((t0, o0, uix), (t1, o1, iix)):
            copies = []
            for j in range(NCH):
                copies.append(pltpu.async_copy(
                    tbl.at[idx.at[j]], buf.at[pl.ds(j * CH, CH)], sem))
            for c in copies:
                c.wait()
            pltpu.sync_copy(buf, out.at[pl.ds(base, BPW)])

    return k(ju2, ji2, pu, pi)


def _select(block, res, k0):
    acc = jnp.zeros((block.shape[0], D), jnp.float32)
    for a in range(PACK):
        acc = acc + jnp.where(res == a, block[:, a * D:(a + 1) * D], 0.0)
    del k0
    return acc


def _unpack(block):
    v = jax.lax.bitcast_convert_type(block, jnp.uint32)
    hi = jax.lax.bitcast_convert_type(v & jnp.uint32(0xFFFF0000),
                                      jnp.float32)
    lo = jax.lax.bitcast_convert_type(v << 16, jnp.float32)
    return hi, lo


def _dense_body(gu_ref, gi_ref, ru_ref, ri_ref,
                w0u, w0i, b0, w1, b1, w2, b2, w3, b3, wpg, wph, bp,
                out_ref):
    f32 = jnp.float32
    ru = ru_ref[...]
    ri = ri_ref[...]
    gug, gum = _unpack(gu_ref[...])
    gig, gim = _unpack(gi_ref[...])
    ug = _select(gug, ru, 0)
    ig = _select(gig, ri, 0)
    um = _select(gum, ru, 0)
    im = _select(gim, ri, 0)
    nu = jnp.sqrt(jnp.sum(ug * ug, axis=1, keepdims=True))
    ni = jnp.sqrt(jnp.sum(ig * ig, axis=1, keepdims=True))
    gmf = (ug / jnp.maximum(nu, 1e-12)) * (ig / jnp.maximum(ni, 1e-12))
    h = (jnp.dot(um, w0u[...], preferred_element_type=f32)
         + jnp.dot(im, w0i[...], preferred_element_type=f32)
         + b0[...])
    h = jnp.maximum(h, 0.0)
    for w, b in ((w1, b1), (w2, b2), (w3, b3)):
        h = jnp.dot(h, w[...], preferred_element_type=f32) + b[...]
        h = jnp.maximum(h, 0.0)
    pred = (jnp.dot(gmf, wpg[...], preferred_element_type=f32)
            + jnp.dot(h, wph[...], preferred_element_type=f32)
            + bp[...])
    out_ref[...] = jax.nn.sigmoid(pred)


def kernel(user_indices, item_indices, user_emb_gmf, item_emb_gmf,
           user_emb_mlp, item_emb_mlp,
           W0, b0, gamma0, beta0, W1, b1, gamma1, beta1,
           W2, b2, gamma2, beta2, W3, b3, gamma3, beta3,
           Wp, bp):
    uidx = user_indices.astype(jnp.int32)
    iidx = item_indices.astype(jnp.int32)
    ju = RB * (uidx // UB) + uidx % RB
    ji = RB * (iidx // UB) + iidx % RB
    ju2 = ju.reshape(BATCH // CH, CH)
    ji2 = ji.reshape(BATCH // CH, CH)
    ru = ((uidx // RB) % PACK).astype(jnp.int32).reshape(BATCH, 1)
    ri = ((iidx // RB) % PACK).astype(jnp.int32).reshape(BATCH, 1)

    pu, pi = _tc_repack((user_emb_gmf.T, user_emb_mlp.T,
                         item_emb_gmf.T, item_emb_mlp.T))
    gu, gi = _sc_gather(ju2, ji2, pu, pi)

    bn = 1.0 / jnp.sqrt(1.0 + BN_EPS)
    def fold(W, b, g, be):
        s = g * bn
        return W * s[None, :], (b * s + be)[None, :]
    W0f, b0f = fold(W0, b0, gamma0, beta0)
    W1f, b1f = fold(W1, b1, gamma1, beta1)
    W2f, b2f = fold(W2, b2, gamma2, beta2)
    W3f, b3f = fold(W3, b3, gamma3, beta3)
    w0u, w0i = W0f[:D], W0f[D:]
    wpg, wph = Wp[:D], Wp[D:]
    bp2 = bp[None, :]

    BB = 2048
    grid = (BATCH // BB,)
    g_spec = pl.BlockSpec((BB, PACK * D), lambda i: (i, 0))
    r_spec = pl.BlockSpec((BB, 1), lambda i: (i, 0))
    def w_spec(a):
        return pl.BlockSpec(a.shape, lambda i: (0,) * a.ndim)
    weights = (w0u, w0i, b0f, W1f, b1f, W2f, b2f, W3f, b3f, wpg, wph, bp2)

    out = pl.pallas_call(
        _dense_body,
        grid=grid,
        in_specs=[g_spec] * 2 + [r_spec] * 2 + [w_spec(a) for a in weights],
        out_specs=pl.BlockSpec((BB, 1), lambda i: (i, 0)),
        out_shape=jax.ShapeDtypeStruct((BATCH, 1), jnp.float32),
    )(gu, gi, ru, ri, *weights)
    return out
